# baseline (speedup 1.0000x reference)
import jax
import jax.numpy as jnp
from jax import lax
from jax.experimental import pallas as pl
from jax.experimental.pallas import tpu as pltpu

_VMEM_LIMIT = 64 * 1024 * 1024


def _xform_kernel(x_ref, w_ref, dis_ref, y_ref):
    xw = jnp.dot(x_ref[...], w_ref[...], preferred_element_type=jnp.float32)
    y_ref[...] = (dis_ref[...] * xw).astype(y_ref.dtype)


def _agg_kernel(a_ref, y_ref, dis_ref, b_ref, o_ref, acc_ref):
    i = pl.program_id(0)
    k = pl.program_id(1)

    @pl.when(k == 0)
    def _():
        acc_ref[...] = jnp.zeros_like(acc_ref)

    y = y_ref[...]
    acc_ref[...] += jnp.dot(a_ref[...], y, preferred_element_type=jnp.float32)

    @pl.when(k == i)
    def _():
        acc_ref[...] += y.astype(jnp.float32)

    @pl.when(k == pl.num_programs(1) - 1)
    def _():
        o_ref[...] = dis_ref[...] * acc_ref[...] + b_ref[...]


def kernel(x, edge_index, weight, bias):
    n, nfeat = x.shape
    nhid = weight.shape[1]
    tm = 512

    src = edge_index[0]
    dst = edge_index[1]

    a = jnp.zeros((n, n), jnp.bfloat16).at[dst, src].add(jnp.bfloat16(1.0))

    deg = jnp.zeros((n,), jnp.float32).at[dst].add(1.0) + 1.0
    d_is = lax.rsqrt(deg)[:, None]

    x_b = x.astype(jnp.bfloat16)
    w_b = weight.astype(jnp.bfloat16)
    b_p = bias.astype(jnp.float32)[None, :]

    y = pl.pallas_call(
        _xform_kernel,
        out_shape=jax.ShapeDtypeStruct((n, nhid), jnp.bfloat16),
        grid=(n // tm,),
        in_specs=[
            pl.BlockSpec((tm, nfeat), lambda i: (i, 0)),
            pl.BlockSpec((nfeat, nhid), lambda i: (0, 0)),
            pl.BlockSpec((tm, 1), lambda i: (i, 0)),
        ],
        out_specs=pl.BlockSpec((tm, nhid), lambda i: (i, 0)),
        compiler_params=pltpu.CompilerParams(
            dimension_semantics=("parallel",),
            vmem_limit_bytes=_VMEM_LIMIT),
    )(x_b, w_b, d_is)

    out = pl.pallas_call(
        _agg_kernel,
        out_shape=jax.ShapeDtypeStruct((n, nhid), jnp.float32),
        grid=(n // tm, n // tm),
        in_specs=[
            pl.BlockSpec((tm, tm), lambda i, k: (i, k)),
            pl.BlockSpec((tm, nhid), lambda i, k: (k, 0)),
            pl.BlockSpec((tm, 1), lambda i, k: (i, 0)),
            pl.BlockSpec((1, nhid), lambda i, k: (0, 0)),
        ],
        out_specs=pl.BlockSpec((tm, nhid), lambda i, k: (i, 0)),
        scratch_shapes=[pltpu.VMEM((tm, nhid), jnp.float32)],
        compiler_params=pltpu.CompilerParams(
            dimension_semantics=("parallel", "arbitrary"),
            vmem_limit_bytes=_VMEM_LIMIT),
    )(a, y, d_is, b_p)

    return out
